# baseline (speedup 1.0000x reference)
import jax
import jax.numpy as jnp
from jax import lax
from jax.experimental import pallas as pl
from jax.experimental.pallas import tpu as pltpu
from jax.experimental.pallas import tpu_sc as plsc

NC = 2
NS = 16
NW = NC * NS
EC = 128
KB = 8
CPAD = 32
ATRASH = 128
CTRASH = 128


def _stage1_prelu_mask(x, mask_idx_2d, prelu_a_2d, blk):
    n, d = x.shape
    grid = n // blk

    def body(x_ref, m_ref, a_ref, o_ref):
        i = pl.program_id(0)
        xb = x_ref[...]
        a = a_ref[0, 0]
        pr = jnp.where(xb >= 0.0, xb, a * xb)
        ids = i * blk + lax.broadcasted_iota(jnp.int32, (blk, 1), 0)
        hit = jnp.any(ids == m_ref[...], axis=1, keepdims=True)
        o_ref[...] = jnp.where(hit, 0.0, pr)

    return pl.pallas_call(
        body,
        grid=(grid,),
        in_specs=[
            pl.BlockSpec((blk, d), lambda i: (i, 0)),
            pl.BlockSpec(mask_idx_2d.shape, lambda i: (0, 0)),
            pl.BlockSpec((1, 1), lambda i: (0, 0), memory_space=pltpu.SMEM),
        ],
        out_specs=pl.BlockSpec((blk, d), lambda i: (i, 0)),
        out_shape=jax.ShapeDtypeStruct((n, d), jnp.float32),
    )(x, mask_idx_2d, prelu_a_2d)


def _edge_attr_prep(edge_attr, nchunks):
    ea2 = edge_attr.reshape(nchunks, 2 * EC).astype(jnp.float32)
    lane = jnp.arange(EC)
    S = (jnp.zeros((2 * EC, EC), jnp.float32)
         .at[2 * lane, lane].set(3.0)
         .at[2 * lane + 1, lane].set(1.0))

    def body(ea_ref, s_ref, o_ref):
        c = jnp.dot(ea_ref[...], s_ref[...], preferred_element_type=jnp.float32)
        o_ref[...] = jnp.clip(c.astype(jnp.int32), 0, CPAD - 1)

    return pl.pallas_call(
        body,
        in_specs=[pl.BlockSpec((nchunks, 2 * EC), lambda: (0, 0)),
                  pl.BlockSpec((2 * EC, EC), lambda: (0, 0))],
        out_specs=pl.BlockSpec((nchunks, EC), lambda: (0, 0)),
        out_shape=jax.ShapeDtypeStruct((nchunks, EC), edge_attr.dtype),
    )(ea2, S)


def _sc_plan(n, e):
    nchunks = e // EC
    per_w = -(--(-nchunks // NW) // KB) * KB
    npad = n + ATRASH
    cnt_w = CPAD * (n + CTRASH)
    cnt_t = (cnt_w // NS) // 1024 * 1024
    cnt_last = cnt_w - cnt_t * (NS - 1)
    rows_t = (n // NS) // 8 * 8
    tail = n - rows_t * NS
    nrows_pad = NW * per_w
    return dict(nchunks=nchunks, per_w=per_w, npad=npad, cnt_w=cnt_w,
                cnt_t=cnt_t, cnt_last=cnt_last, rows_t=rows_t, tail=tail,
                nrows_pad=nrows_pad)


def _make_sc_kernel(n, d, e):
    pp = _sc_plan(n, e)
    per_w, npad, cnt_w = pp["per_w"], pp["npad"], pp["cnt_w"]
    cnt_t, cnt_last = pp["cnt_t"], pp["cnt_last"]
    rows_t, tail = pp["rows_t"], pp["tail"]
    ZC = 512
    n_zfull = rows_t // EC
    zrem = rows_t - n_zfull * EC
    n_blk = per_w // KB

    mesh = plsc.VectorSubcoreMesh(core_axis_name="c", subcore_axis_name="s")

    def body(p_hbm, src_hbm, dst_hbm, c_hbm, aggr_out, cnt0_out,
             cnt1_out, src_v, dst_v, c_v, src_d, dst_d, fidx_d,
             rows_v, ones_v, zc_v, sem, aggr_sp, cnt_sp):
        cid = lax.axis_index("c")
        sid = lax.axis_index("s")
        wid = sid * NC + cid

        zero16 = jnp.zeros((16,), jnp.float32)
        one16 = jnp.ones((16,), jnp.float32)

        def zrow(r, _):
            for c in range(d // 16):
                rows_v[r, pl.ds(16 * c, 16)] = zero16
            return 0
        lax.fori_loop(0, EC, zrow, 0)

        for i in range(ZC // 16):
            zc_v[pl.ds(i * 16, 16)] = zero16

        for k in range(EC // 16):
            ones_v[pl.ds(16 * k, 16)] = one16

        row0 = sid * rows_t
        for k in range(n_zfull):
            pltpu.sync_copy(rows_v, aggr_sp.at[pl.ds(row0 + k * EC, EC)])
        if zrem:
            pltpu.sync_copy(rows_v.at[pl.ds(0, zrem)],
                            aggr_sp.at[pl.ds(row0 + n_zfull * EC, zrem)])
        if tail:
            @pl.when(sid == NS - 1)
            def _():
                pltpu.sync_copy(rows_v.at[pl.ds(0, tail)],
                                aggr_sp.at[pl.ds(NS * rows_t, tail)])
        off0 = sid * cnt_t
        for k in range(cnt_t // ZC):
            pltpu.sync_copy(zc_v, cnt_sp.at[pl.ds(off0 + k * ZC, ZC)])

        @pl.when(sid == NS - 1)
        def _():
            base = NS * cnt_t
            for k in range((cnt_last - cnt_t) // ZC):
                pltpu.sync_copy(zc_v, cnt_sp.at[pl.ds(base + k * ZC, ZC)])

        plsc.subcore_barrier()

        def chunk(q, _):
            for k in range(EC // 16):
                sl = pl.ds(16 * k, 16)
                dd = dst_v[q, sl]
                src_d[sl] = src_v[q, sl]
                dst_d[sl] = dd
                fidx_d[sl] = dd * CPAD + c_v[q, sl]
            pltpu.async_copy(p_hbm.at[src_d], rows_v, sem).wait()
            pltpu.sync_copy(rows_v, aggr_sp.at[dst_d], add=True)
            pltpu.sync_copy(ones_v, cnt_sp.at[fidx_d], add=True)
            return 0

        def blk(j, _):
            b0 = (wid + NW * j) * KB
            pltpu.sync_copy(src_hbm.at[pl.ds(b0, KB)], src_v)
            pltpu.sync_copy(dst_hbm.at[pl.ds(b0, KB)], dst_v)
            pltpu.sync_copy(c_hbm.at[pl.ds(b0, KB)], c_v)
            lax.fori_loop(0, KB, chunk, 0)
            return 0

        lax.fori_loop(0, n_blk, blk, 0)

        plsc.subcore_barrier()

        pltpu.sync_copy(aggr_sp.at[pl.ds(row0, rows_t)],
                        aggr_out.at[cid, pl.ds(row0, rows_t)])
        if tail:
            @pl.when(sid == NS - 1)
            def _():
                pltpu.sync_copy(aggr_sp.at[pl.ds(NS * rows_t, tail)],
                                aggr_out.at[cid, pl.ds(NS * rows_t, tail)])

        def wr_cnt(cnt_out):
            pltpu.sync_copy(cnt_sp.at[pl.ds(off0, cnt_t)],
                            cnt_out.at[pl.ds(off0, cnt_t)])

            @pl.when(sid == NS - 1)
            def _():
                base = NS * cnt_t
                pltpu.sync_copy(cnt_sp.at[pl.ds(base, cnt_last - cnt_t)],
                                cnt_out.at[pl.ds(base, cnt_last - cnt_t)])

        @pl.when(cid == 0)
        def _():
            wr_cnt(cnt0_out)

        @pl.when(cid == 1)
        def _():
            wr_cnt(cnt1_out)

    return pl.kernel(
        body,
        out_type=[
            jax.ShapeDtypeStruct((NC, n, d), jnp.float32),
            jax.ShapeDtypeStruct((cnt_w,), jnp.float32),
            jax.ShapeDtypeStruct((cnt_w,), jnp.float32),
        ],
        mesh=mesh,
        scratch_types=[
            pltpu.VMEM((KB, EC), jnp.int32),
            pltpu.VMEM((KB, EC), jnp.int32),
            pltpu.VMEM((KB, EC), jnp.int32),
            pltpu.VMEM((EC,), jnp.int32),
            pltpu.VMEM((EC,), jnp.int32),
            pltpu.VMEM((EC,), jnp.int32),
            pltpu.VMEM((EC, d), jnp.float32),
            pltpu.VMEM((EC,), jnp.float32),
            pltpu.VMEM((ZC,), jnp.float32),
            pltpu.SemaphoreType.DMA,
            pltpu.VMEM_SHARED((npad, d), jnp.float32),
            pltpu.VMEM_SHARED((cnt_w,), jnp.float32),
        ],
    )


def _stage3_mlp(aggr, p, cA, cB, W_enc, Tpad, W1, b1_2d, W2, b2_2d, blk):
    n, d = p.shape
    dh = W1.shape[0]
    grid = n // blk
    f32 = jnp.float32

    def body(g_ref, p_ref, ca_ref, cb_ref, we_ref, t_ref, w1_ref, b1_ref,
             w2_ref, b2_ref, o_ref):
        g = g_ref[0] + g_ref[1] + p_ref[...]
        acc = lax.dot_general(g, we_ref[...], (((1,), (1,)), ((), ())),
                              preferred_element_type=f32)
        c = ca_ref[...] + cb_ref[...]
        acc = acc + jnp.dot(c, t_ref[...], preferred_element_type=f32)
        acc = acc + t_ref[12:13, :]
        h1 = lax.dot_general(acc, w1_ref[...], (((1,), (1,)), ((), ())),
                             preferred_element_type=f32) + b1_ref[...]
        h1 = jnp.maximum(h1, 0.0)
        o_ref[...] = lax.dot_general(h1, w2_ref[...], (((1,), (1,)), ((), ())),
                                     preferred_element_type=f32) + b2_ref[...]

    return pl.pallas_call(
        body,
        grid=(grid,),
        in_specs=[
            pl.BlockSpec((NC, blk, d), lambda i: (0, i, 0)),
            pl.BlockSpec((blk, d), lambda i: (i, 0)),
            pl.BlockSpec((blk, CPAD), lambda i: (i, 0)),
            pl.BlockSpec((blk, CPAD), lambda i: (i, 0)),
            pl.BlockSpec((d, d), lambda i: (0, 0)),
            pl.BlockSpec((CPAD, d), lambda i: (0, 0)),
            pl.BlockSpec((dh, d), lambda i: (0, 0)),
            pl.BlockSpec((1, dh), lambda i: (0, 0)),
            pl.BlockSpec((d, dh), lambda i: (0, 0)),
            pl.BlockSpec((1, d), lambda i: (0, 0)),
        ],
        out_specs=pl.BlockSpec((blk, d), lambda i: (i, 0)),
        out_shape=jax.ShapeDtypeStruct((n, d), jnp.float32),
    )(aggr, p, cA, cB, W_enc, Tpad, W1, b1_2d, W2, b2_2d)


def kernel(x, edge_index, edge_attr, mask_node_indices, prelu_a, W_enc,
           emb1, emb2, W1, b1, W2, b2):
    n, d = x.shape
    e = edge_index.shape[1]
    nm = mask_node_indices.shape[0]

    padw = -(-nm // 128) * 128
    m2d = jnp.concatenate(
        [mask_node_indices,
         jnp.full((padw - nm,), -1, mask_node_indices.dtype)]).reshape(1, padw)
    a2d = prelu_a.reshape(1, 1)
    pp = _sc_plan(n, e)
    nchunks = pp["nchunks"]
    padr = pp["nrows_pad"] - nchunks
    spread = jnp.arange(padr * EC, dtype=edge_index.dtype) % ATRASH
    src1 = jnp.concatenate([edge_index[0].reshape(nchunks, EC),
                            spread.reshape(padr, EC)])
    trash = (n + spread).reshape(padr, EC)
    dst1 = jnp.concatenate([edge_index[1].reshape(nchunks, EC), trash])
    c1 = jnp.pad(_edge_attr_prep(edge_attr, nchunks), ((0, padr), (0, 0)))
    T = (emb1[:, None, :] + emb2[None, :, :]).reshape(-1, d)
    Tpad = jnp.concatenate([T, jnp.zeros((CPAD - T.shape[0], d), T.dtype)])
    b1_2d = b1.reshape(1, -1)
    b2_2d = b2.reshape(1, -1)

    p = _stage1_prelu_mask(x, m2d, a2d, blk=1000)

    aggr, cnt0, cnt1 = _make_sc_kernel(n, d, e)(p, src1, dst1, c1)
    cA = cnt0.reshape(n + CTRASH, CPAD)
    cB = cnt1.reshape(n + CTRASH, CPAD)

    return _stage3_mlp(aggr, p, cA, cB, W_enc, Tpad, W1, b1_2d, W2, b2_2d,
                       blk=1000)


# speedup vs baseline: 1.6159x; 1.6159x over previous
import jax
import jax.numpy as jnp
from jax import lax
from jax.experimental import pallas as pl
from jax.experimental.pallas import tpu as pltpu
from jax.experimental.pallas import tpu_sc as plsc

NC = 2
NS = 16
NW = NC * NS
EC = 128
KB = 8
CPAD = 32
ATRASH = 128
CTRASH = 128


def _stage1_prelu_mask(x, mask_idx_2d, prelu_a_2d, blk):
    n, d = x.shape
    grid = n // blk

    def body(x_ref, m_ref, a_ref, o_ref):
        i = pl.program_id(0)
        xb = x_ref[...]
        a = a_ref[0, 0]
        pr = jnp.where(xb >= 0.0, xb, a * xb)
        ids = i * blk + lax.broadcasted_iota(jnp.int32, (blk, 1), 0)
        hit = jnp.any(ids == m_ref[...], axis=1, keepdims=True)
        o_ref[...] = jnp.where(hit, 0.0, pr)

    return pl.pallas_call(
        body,
        grid=(grid,),
        in_specs=[
            pl.BlockSpec((blk, d), lambda i: (i, 0)),
            pl.BlockSpec(mask_idx_2d.shape, lambda i: (0, 0)),
            pl.BlockSpec((1, 1), lambda i: (0, 0), memory_space=pltpu.SMEM),
        ],
        out_specs=pl.BlockSpec((blk, d), lambda i: (i, 0)),
        out_shape=jax.ShapeDtypeStruct((n, d), jnp.float32),
    )(x, mask_idx_2d, prelu_a_2d)


def _sc_plan(n, e):
    nchunks = e // EC
    per_w = -(--(-nchunks // NW) // KB) * KB
    npad = n + ATRASH
    cnt_w = CPAD * (n + CTRASH)
    cnt_t = (cnt_w // NS) // 1024 * 1024
    cnt_last = cnt_w - cnt_t * (NS - 1)
    rows_t = (n // NS) // 8 * 8
    tail = n - rows_t * NS
    nrows_pad = NW * per_w
    return dict(nchunks=nchunks, per_w=per_w, npad=npad, cnt_w=cnt_w,
                cnt_t=cnt_t, cnt_last=cnt_last, rows_t=rows_t, tail=tail,
                nrows_pad=nrows_pad)


def _make_sc_kernel(n, d, e):
    pp = _sc_plan(n, e)
    per_w, npad, cnt_w = pp["per_w"], pp["npad"], pp["cnt_w"]
    cnt_t, cnt_last = pp["cnt_t"], pp["cnt_last"]
    rows_t, tail = pp["rows_t"], pp["tail"]
    ZC = 512
    n_zfull = rows_t // EC
    zrem = rows_t - n_zfull * EC
    n_blk = per_w // KB

    mesh = plsc.VectorSubcoreMesh(core_axis_name="c", subcore_axis_name="s")

    def body(p_hbm, src_hbm, dst_hbm, a0_hbm, a1_hbm, aggr_out, cnt0_out,
             cnt1_out, src_v, dst_v, a0_v, a1_v, src_d, dst_d, fidx_d,
             rows_v, ones_v, zc_v, sem, aggr_sp, cnt_sp):
        cid = lax.axis_index("c")
        sid = lax.axis_index("s")
        wid = sid * NC + cid

        zero16 = jnp.zeros((16,), jnp.float32)
        one16 = jnp.ones((16,), jnp.float32)

        def zrow(r, _):
            for c in range(d // 16):
                rows_v[r, pl.ds(16 * c, 16)] = zero16
            return 0
        lax.fori_loop(0, EC, zrow, 0)

        for i in range(ZC // 16):
            zc_v[pl.ds(i * 16, 16)] = zero16

        for k in range(EC // 16):
            ones_v[pl.ds(16 * k, 16)] = one16

        row0 = sid * rows_t
        for k in range(n_zfull):
            pltpu.sync_copy(rows_v, aggr_sp.at[pl.ds(row0 + k * EC, EC)])
        if zrem:
            pltpu.sync_copy(rows_v.at[pl.ds(0, zrem)],
                            aggr_sp.at[pl.ds(row0 + n_zfull * EC, zrem)])
        if tail:
            @pl.when(sid == NS - 1)
            def _():
                pltpu.sync_copy(rows_v.at[pl.ds(0, tail)],
                                aggr_sp.at[pl.ds(NS * rows_t, tail)])
        off0 = sid * cnt_t
        for k in range(cnt_t // ZC):
            pltpu.sync_copy(zc_v, cnt_sp.at[pl.ds(off0 + k * ZC, ZC)])

        @pl.when(sid == NS - 1)
        def _():
            base = NS * cnt_t
            for k in range((cnt_last - cnt_t) // ZC):
                pltpu.sync_copy(zc_v, cnt_sp.at[pl.ds(base + k * ZC, ZC)])

        plsc.subcore_barrier()

        def chunk(q, _):
            for k in range(EC // 16):
                sl = pl.ds(16 * k, 16)
                dd = dst_v[q, sl]
                aa = jnp.minimum(jnp.maximum(a0_v[q, sl], 0), 5)
                bb = jnp.minimum(jnp.maximum(a1_v[q, sl], 0), 2)
                src_d[sl] = src_v[q, sl]
                dst_d[sl] = dd
                fidx_d[sl] = dd * CPAD + aa * 3 + bb
            pltpu.async_copy(p_hbm.at[src_d], rows_v, sem).wait()
            pltpu.sync_copy(rows_v, aggr_sp.at[dst_d], add=True)
            pltpu.sync_copy(ones_v, cnt_sp.at[fidx_d], add=True)
            return 0

        def blk(j, _):
            b0 = (wid + NW * j) * KB
            pltpu.sync_copy(src_hbm.at[pl.ds(b0, KB)], src_v)
            pltpu.sync_copy(dst_hbm.at[pl.ds(b0, KB)], dst_v)
            pltpu.sync_copy(a0_hbm.at[pl.ds(b0, KB)], a0_v)
            pltpu.sync_copy(a1_hbm.at[pl.ds(b0, KB)], a1_v)
            lax.fori_loop(0, KB, chunk, 0)
            return 0

        lax.fori_loop(0, n_blk, blk, 0)

        plsc.subcore_barrier()

        pltpu.sync_copy(aggr_sp.at[pl.ds(row0, rows_t)],
                        aggr_out.at[cid, pl.ds(row0, rows_t)])
        if tail:
            @pl.when(sid == NS - 1)
            def _():
                pltpu.sync_copy(aggr_sp.at[pl.ds(NS * rows_t, tail)],
                                aggr_out.at[cid, pl.ds(NS * rows_t, tail)])

        def wr_cnt(cnt_out):
            pltpu.sync_copy(cnt_sp.at[pl.ds(off0, cnt_t)],
                            cnt_out.at[pl.ds(off0, cnt_t)])

            @pl.when(sid == NS - 1)
            def _():
                base = NS * cnt_t
                pltpu.sync_copy(cnt_sp.at[pl.ds(base, cnt_last - cnt_t)],
                                cnt_out.at[pl.ds(base, cnt_last - cnt_t)])

        @pl.when(cid == 0)
        def _():
            wr_cnt(cnt0_out)

        @pl.when(cid == 1)
        def _():
            wr_cnt(cnt1_out)

    return pl.kernel(
        body,
        out_type=[
            jax.ShapeDtypeStruct((NC, n, d), jnp.float32),
            jax.ShapeDtypeStruct((cnt_w,), jnp.float32),
            jax.ShapeDtypeStruct((cnt_w,), jnp.float32),
        ],
        mesh=mesh,
        scratch_types=[
            pltpu.VMEM((KB, EC), jnp.int32),
            pltpu.VMEM((KB, EC), jnp.int32),
            pltpu.VMEM((KB, EC), jnp.int32),
            pltpu.VMEM((KB, EC), jnp.int32),
            pltpu.VMEM((EC,), jnp.int32),
            pltpu.VMEM((EC,), jnp.int32),
            pltpu.VMEM((EC,), jnp.int32),
            pltpu.VMEM((EC, d), jnp.float32),
            pltpu.VMEM((EC,), jnp.float32),
            pltpu.VMEM((ZC,), jnp.float32),
            pltpu.SemaphoreType.DMA,
            pltpu.VMEM_SHARED((npad, d), jnp.float32),
            pltpu.VMEM_SHARED((cnt_w,), jnp.float32),
        ],
    )


def _stage3_mlp(aggr, p, cA, cB, W_enc, Tpad, W1, b1_2d, W2, b2_2d, blk):
    n, d = p.shape
    dh = W1.shape[0]
    grid = n // blk
    f32 = jnp.float32

    def body(g_ref, p_ref, ca_ref, cb_ref, we_ref, t_ref, w1_ref, b1_ref,
             w2_ref, b2_ref, o_ref):
        g = g_ref[0] + g_ref[1] + p_ref[...]
        acc = lax.dot_general(g, we_ref[...], (((1,), (1,)), ((), ())),
                              preferred_element_type=f32)
        c = ca_ref[...] + cb_ref[...]
        acc = acc + jnp.dot(c, t_ref[...], preferred_element_type=f32)
        acc = acc + t_ref[12:13, :]
        h1 = lax.dot_general(acc, w1_ref[...], (((1,), (1,)), ((), ())),
                             preferred_element_type=f32) + b1_ref[...]
        h1 = jnp.maximum(h1, 0.0)
        o_ref[...] = lax.dot_general(h1, w2_ref[...], (((1,), (1,)), ((), ())),
                                     preferred_element_type=f32) + b2_ref[...]

    return pl.pallas_call(
        body,
        grid=(grid,),
        in_specs=[
            pl.BlockSpec((NC, blk, d), lambda i: (0, i, 0)),
            pl.BlockSpec((blk, d), lambda i: (i, 0)),
            pl.BlockSpec((blk, CPAD), lambda i: (i, 0)),
            pl.BlockSpec((blk, CPAD), lambda i: (i, 0)),
            pl.BlockSpec((d, d), lambda i: (0, 0)),
            pl.BlockSpec((CPAD, d), lambda i: (0, 0)),
            pl.BlockSpec((dh, d), lambda i: (0, 0)),
            pl.BlockSpec((1, dh), lambda i: (0, 0)),
            pl.BlockSpec((d, dh), lambda i: (0, 0)),
            pl.BlockSpec((1, d), lambda i: (0, 0)),
        ],
        out_specs=pl.BlockSpec((blk, d), lambda i: (i, 0)),
        out_shape=jax.ShapeDtypeStruct((n, d), jnp.float32),
    )(aggr, p, cA, cB, W_enc, Tpad, W1, b1_2d, W2, b2_2d)


def kernel(x, edge_index, edge_attr, mask_node_indices, prelu_a, W_enc,
           emb1, emb2, W1, b1, W2, b2):
    n, d = x.shape
    e = edge_index.shape[1]
    nm = mask_node_indices.shape[0]

    padw = -(-nm // 128) * 128
    m2d = jnp.concatenate(
        [mask_node_indices,
         jnp.full((padw - nm,), -1, mask_node_indices.dtype)]).reshape(1, padw)
    a2d = prelu_a.reshape(1, 1)
    pp = _sc_plan(n, e)
    nchunks = pp["nchunks"]
    padr = pp["nrows_pad"] - nchunks
    spread = jnp.arange(padr * EC, dtype=edge_index.dtype) % ATRASH
    src1 = jnp.concatenate([edge_index[0].reshape(nchunks, EC),
                            spread.reshape(padr, EC)])
    trash = (n + spread).reshape(padr, EC)
    dst1 = jnp.concatenate([edge_index[1].reshape(nchunks, EC), trash])
    a01 = jnp.pad(edge_attr[:, 0].reshape(nchunks, EC), ((0, padr), (0, 0)))
    a11 = jnp.pad(edge_attr[:, 1].reshape(nchunks, EC), ((0, padr), (0, 0)))
    T = (emb1[:, None, :] + emb2[None, :, :]).reshape(-1, d)
    Tpad = jnp.concatenate([T, jnp.zeros((CPAD - T.shape[0], d), T.dtype)])
    b1_2d = b1.reshape(1, -1)
    b2_2d = b2.reshape(1, -1)

    p = _stage1_prelu_mask(x, m2d, a2d, blk=1000)

    aggr, cnt0, cnt1 = _make_sc_kernel(n, d, e)(p, src1, dst1, a01, a11)
    cA = cnt0.reshape(n + CTRASH, CPAD)
    cB = cnt1.reshape(n + CTRASH, CPAD)

    return _stage3_mlp(aggr, p, cA, cB, W_enc, Tpad, W1, b1_2d, W2, b2_2d,
                       blk=1000)
